# baseline (speedup 1.0000x reference)
import functools

import jax
import jax.numpy as jnp
from jax import lax
from jax.experimental import pallas as pl
from jax.experimental.pallas import tpu as pltpu
from jax.experimental.pallas import tpu_sc as plsc

_B = 16
_P = 128
_T = 512
_SRC_LEN = 8
_PATH_LEN = 12
_IN_DIM = 128
_H_DIM = 256
_NUM_LAYERS = 2
_NW = 32


def _sc_ctx_gather(ctx_i, ctx_t):
    ctx_per_w = (_B * _T) // _NW
    mesh = plsc.VectorSubcoreMesh(core_axis_name="c", subcore_axis_name="s")

    @functools.partial(
        pl.kernel,
        out_type=jax.ShapeDtypeStruct((_B * _T, _IN_DIM), jnp.float32),
        mesh=mesh,
        scratch_types=[
            pltpu.VMEM((ctx_per_w,), jnp.int32),
            pltpu.VMEM((ctx_per_w, _IN_DIM), jnp.float32),
            pltpu.SemaphoreType.DMA,
        ],
    )
    def k(ctxi_r, ctxt_r, ctx_o, idxc, rows, sem):
        wid = lax.axis_index("s") * 2 + lax.axis_index("c")
        cb = wid * ctx_per_w
        pltpu.sync_copy(ctxi_r.at[pl.ds(cb, ctx_per_w)], idxc)
        pltpu.async_copy(ctxt_r.at[idxc], rows, sem).wait()
        pltpu.sync_copy(rows, ctx_o.at[pl.ds(cb, ctx_per_w)])

    return k(ctx_i, ctx_t)


def _sc_path_sums(srcs_i, tgts_i, paths_i, st_t, path_t):
    n_rows = _B * _P
    rows_per_w = n_rows // _NW
    mesh = plsc.VectorSubcoreMesh(core_axis_name="c", subcore_axis_name="s")

    @functools.partial(
        pl.kernel,
        out_type=(
            jax.ShapeDtypeStruct((n_rows, _IN_DIM), jnp.float32),
            jax.ShapeDtypeStruct((n_rows, _IN_DIM), jnp.float32),
            jax.ShapeDtypeStruct((n_rows, _IN_DIM), jnp.float32),
        ),
        mesh=mesh,
        scratch_types=[
            pltpu.VMEM((16 * _SRC_LEN,), jnp.int32),
            pltpu.VMEM((16 * _PATH_LEN,), jnp.int32),
            pltpu.VMEM((16 * _PATH_LEN, _IN_DIM), jnp.float32),
            pltpu.VMEM((16, _IN_DIM), jnp.float32),
            pltpu.SemaphoreType.DMA,
        ],
    )
    def k(srcs_r, tgts_r, paths_r, st_r, path_r,
          src_o, tgt_o, pth_o, idx8, idx12, rows, acc, sem):
        wid = lax.axis_index("s") * 2 + lax.axis_index("c")

        def reduce_rep(idx_hbm, table, out_hbm, G, idx_v):
            base = wid * rows_per_w

            def chunk(c, carry):
                rb = base + c * 16
                pltpu.sync_copy(idx_hbm.at[pl.ds(rb * G, 16 * G)], idx_v)
                pltpu.async_copy(table.at[idx_v],
                                 rows.at[pl.ds(0, 16 * G)], sem).wait()

                def rowfn(r, carry2):
                    for v in range(_IN_DIM // 16):
                        s = rows[r * G, pl.ds(v * 16, 16)]
                        for j in range(1, G):
                            s = s + rows[r * G + j, pl.ds(v * 16, 16)]
                        acc[r, pl.ds(v * 16, 16)] = s
                    return carry2

                lax.fori_loop(0, 16, rowfn, 0)
                pltpu.sync_copy(acc, out_hbm.at[pl.ds(rb, 16)])
                return carry

            lax.fori_loop(0, rows_per_w // 16, chunk, 0)

        reduce_rep(srcs_r, st_r, src_o, _SRC_LEN, idx8)
        reduce_rep(tgts_r, st_r, tgt_o, _SRC_LEN, idx8)
        reduce_rep(paths_r, path_r, pth_o, _PATH_LEN, idx12)

    return k(srcs_i, tgts_i, paths_i, st_t, path_t)


def _fuse_body(ss, ts, ps, stok, spos, ttok, tpos, ptok, ppos,
               r0st, r0p, pose, wp, bp, wm, bm, csum, dlens, mixed_o, h_o):
    n = _B * _P
    pos_tab = pose[...].astype(jnp.bfloat16)

    def rep(sum_ref, tok_ref, pos_ref, row0_ref, G):
        tok = tok_ref[...]
        posi = pos_ref[...]
        mask = (tok != 0).astype(jnp.float32)
        cnt = mask.sum(axis=1, keepdims=True)
        cnt0 = G - cnt
        iota32 = lax.broadcasted_iota(jnp.int32, (n, 32), 1)
        hist = jnp.zeros((n, 32), jnp.float32)
        for j in range(G):
            hist = hist + (posi[:, j:j + 1] == iota32).astype(jnp.float32) * mask[:, j:j + 1]
        pos_contrib = jnp.dot(hist.astype(jnp.bfloat16), pos_tab,
                              preferred_element_type=jnp.float32)
        return (sum_ref[...] - cnt0 * row0_ref[...] + pos_contrib) / jnp.maximum(cnt, 1.0)

    sr = rep(ss, stok, spos, r0st, _SRC_LEN)
    tr = rep(ts, ttok, tpos, r0st, _SRC_LEN)
    pr = rep(ps, ptok, ppos, r0p, _PATH_LEN)

    wp_v = wp[...].astype(jnp.bfloat16)
    ep = jnp.tanh(
        jnp.dot(sr.astype(jnp.bfloat16), wp_v[0:128], preferred_element_type=jnp.float32)
        + jnp.dot(tr.astype(jnp.bfloat16), wp_v[128:256], preferred_element_type=jnp.float32)
        + jnp.dot(pr.astype(jnp.bfloat16), wp_v[256:384], preferred_element_type=jnp.float32)
        + bp[...])
    mixed_o[...] = jnp.tanh(
        jnp.dot(ep.astype(jnp.bfloat16), wm[...].astype(jnp.bfloat16),
                preferred_element_type=jnp.float32) + bm[...])
    gids = lax.broadcasted_iota(jnp.int32, (_B, n), 1) // _P
    sel = (gids == lax.broadcasted_iota(jnp.int32, (_B, n), 0)).astype(jnp.float32)
    psum = jnp.dot(sel, ep, preferred_element_type=jnp.float32)
    hf = (psum + csum[...]) / dlens[...]
    h_o[...] = jnp.broadcast_to(hf[None], (_NUM_LAYERS, _B, _H_DIM))


def _tc_fuse(src_sum, tgt_sum, pth_sum, stok, spos, ttok, tpos, ptok, ppos,
             r0st, r0p, pos_emb, wp, bp, wm, bm, csum, dlens):
    n = _B * _P
    return pl.pallas_call(
        _fuse_body,
        out_shape=(
            jax.ShapeDtypeStruct((n, _H_DIM), jnp.float32),
            jax.ShapeDtypeStruct((_NUM_LAYERS, _B, _H_DIM), jnp.float32),
        ),
    )(src_sum, tgt_sum, pth_sum, stok, spos, ttok, tpos, ptok, ppos,
      r0st, r0p, pos_emb, wp, bp, wm, bm, csum, dlens)


_TCHUNK = 64


def _gru_body(ctx_ref, wx_ref, wh_ref, bg_ref, len_ref,
              enc_ref, csum_o, h_s, csum_s, xg_s, ys_s):
    i = pl.program_id(0)

    @pl.when(i == 0)
    def _init():
        h_s[...] = jnp.zeros_like(h_s)
        csum_s[...] = jnp.zeros_like(csum_s)

    x = ctx_ref[...].astype(jnp.bfloat16)
    xg_s[...] = jnp.dot(x, wx_ref[...].astype(jnp.bfloat16),
                        preferred_element_type=jnp.float32) + bg_ref[...]
    lens = len_ref[...]
    wh = wh_ref[...].astype(jnp.bfloat16)
    _G = _B // 2

    def gates(xgt, hg, h, t, lens_g):
        xr = xgt[:, 0:_H_DIM]
        xz = xgt[:, _H_DIM:2 * _H_DIM]
        xn = xgt[:, 2 * _H_DIM:]
        hr = hg[:, 0:_H_DIM]
        hz = hg[:, _H_DIM:2 * _H_DIM]
        hn = hg[:, 2 * _H_DIM:]
        r = jax.nn.sigmoid(xr + hr)
        z = jax.nn.sigmoid(xz + hz)
        nn_ = jnp.tanh(xn + r * hn)
        h_new = (1.0 - z) * nn_ + z * h
        m = (i * _TCHUNK + t) < lens_g
        h_out = jnp.where(m, h_new, h)
        return h_out, m

    def step(t, carry):
        h_a, h_b, hg_a, hg_b = carry
        xgt_a = xg_s[pl.ds(t * _B, _G), :]
        ha_out, ma = gates(xgt_a, hg_a, h_a, t, lens[0:_G])
        ys_s[pl.ds(t * _B, _G), :] = ha_out
        csum_s[0:_G, :] = csum_s[0:_G, :] + jnp.where(ma, ha_out, 0.0)
        hg_a_next = jnp.dot(ha_out.astype(jnp.bfloat16), wh, preferred_element_type=jnp.float32)
        xgt_b = xg_s[pl.ds(t * _B + _G, _G), :]
        hb_out, mb = gates(xgt_b, hg_b, h_b, t, lens[_G:])
        ys_s[pl.ds(t * _B + _G, _G), :] = hb_out
        csum_s[_G:, :] = csum_s[_G:, :] + jnp.where(mb, hb_out, 0.0)
        hg_b_next = jnp.dot(hb_out.astype(jnp.bfloat16), wh, preferred_element_type=jnp.float32)
        return ha_out, hb_out, hg_a_next, hg_b_next

    h0 = h_s[...]
    hg0 = jnp.dot(h0.astype(jnp.bfloat16), wh, preferred_element_type=jnp.float32)
    ha_f, hb_f, _, _ = lax.fori_loop(
        0, _TCHUNK, step, (h0[0:_G], h0[_G:], hg0[0:_G], hg0[_G:]))
    h_s[...] = jnp.concatenate([ha_f, hb_f], axis=0)
    enc_ref[...] = ys_s[...].reshape(_TCHUNK, _B, _H_DIM).transpose(1, 0, 2)

    @pl.when(i == (_T // _TCHUNK) - 1)
    def _fin():
        csum_o[...] = csum_s[...]


def _tc_gru(ctx_in_tm, wx, wh, bg, lens):
    nchunks = _T // _TCHUNK
    return pl.pallas_call(
        _gru_body,
        grid=(nchunks,),
        in_specs=[
            pl.BlockSpec((_TCHUNK * _B, _IN_DIM), lambda i: (i, 0)),
            pl.BlockSpec((_IN_DIM, 3 * _H_DIM), lambda i: (0, 0)),
            pl.BlockSpec((_H_DIM, 3 * _H_DIM), lambda i: (0, 0)),
            pl.BlockSpec((1, 3 * _H_DIM), lambda i: (0, 0)),
            pl.BlockSpec((_B, 1), lambda i: (0, 0)),
        ],
        out_specs=[
            pl.BlockSpec((_B, _TCHUNK, _H_DIM), lambda i: (0, i, 0)),
            pl.BlockSpec((_B, _H_DIM), lambda i: (0, 0)),
        ],
        out_shape=(
            jax.ShapeDtypeStruct((_B, _T, _H_DIM), jnp.float32),
            jax.ShapeDtypeStruct((_B, _H_DIM), jnp.float32),
        ),
        scratch_shapes=[
            pltpu.VMEM((_B, _H_DIM), jnp.float32),
            pltpu.VMEM((_B, _H_DIM), jnp.float32),
            pltpu.VMEM((_TCHUNK * _B, 3 * _H_DIM), jnp.float32),
            pltpu.VMEM((_TCHUNK * _B, _H_DIM), jnp.float32),
        ],
    )(ctx_in_tm, wx, wh, bg, lens)


def kernel(packed_srcs, packed_srcs_positions, packed_tgts, packed_tgts_positions,
           packed_paths, packed_paths_positions, packed_ctx, ctx_lengths,
           focus_num_of_paths, path_emb, src_tgt_emb, pos_emb, ctx_table,
           W_path, b_path, W_mix, b_mix, Wx, Wh, bg):
    ctx_rows_tm = _sc_ctx_gather(packed_ctx.T.reshape(-1), ctx_table)

    src_sum, tgt_sum, pth_sum = _sc_path_sums(
        packed_srcs.reshape(-1), packed_tgts.reshape(-1),
        packed_paths.reshape(-1), src_tgt_emb, path_emb)

    lens = ctx_lengths.reshape(_B, 1)
    enc, csum = _tc_gru(ctx_rows_tm, Wx, Wh, bg.reshape(1, -1), lens)

    dlens = (focus_num_of_paths + ctx_lengths).astype(jnp.float32).reshape(_B, 1)
    mixed, h = _tc_fuse(
        src_sum, tgt_sum, pth_sum,
        packed_srcs, packed_srcs_positions,
        packed_tgts, packed_tgts_positions,
        packed_paths, packed_paths_positions,
        src_tgt_emb[0:1], path_emb[0:1], pos_emb,
        W_path, b_path.reshape(1, -1), W_mix, b_mix.reshape(1, -1),
        csum, dlens)

    return (mixed.reshape(_B, _P, _H_DIM), enc, h)


# speedup vs baseline: 1.0277x; 1.0277x over previous
import functools

import jax
import jax.numpy as jnp
from jax import lax
from jax.experimental import pallas as pl
from jax.experimental.pallas import tpu as pltpu
from jax.experimental.pallas import tpu_sc as plsc

_B = 16
_P = 128
_T = 512
_SRC_LEN = 8
_PATH_LEN = 12
_IN_DIM = 128
_H_DIM = 256
_NUM_LAYERS = 2
_NW = 32


def _sc_ctx_gather(ctx_i, ctx_t):
    ctx_per_w = (_B * _T) // _NW
    mesh = plsc.VectorSubcoreMesh(core_axis_name="c", subcore_axis_name="s")

    @functools.partial(
        pl.kernel,
        out_type=jax.ShapeDtypeStruct((_B * _T, _IN_DIM), jnp.float32),
        mesh=mesh,
        scratch_types=[
            pltpu.VMEM((ctx_per_w,), jnp.int32),
            pltpu.VMEM((ctx_per_w, _IN_DIM), jnp.float32),
            pltpu.SemaphoreType.DMA,
        ],
    )
    def k(ctxi_r, ctxt_r, ctx_o, idxc, rows, sem):
        wid = lax.axis_index("s") * 2 + lax.axis_index("c")
        cb = wid * ctx_per_w
        pltpu.sync_copy(ctxi_r.at[pl.ds(cb, ctx_per_w)], idxc)
        pltpu.async_copy(ctxt_r.at[idxc], rows, sem).wait()
        pltpu.sync_copy(rows, ctx_o.at[pl.ds(cb, ctx_per_w)])

    return k(ctx_i, ctx_t)


def _sc_path_sums(srcs_i, tgts_i, paths_i, st_t, path_t):
    n_rows = _B * _P
    rows_per_w = n_rows // _NW
    mesh = plsc.VectorSubcoreMesh(core_axis_name="c", subcore_axis_name="s")

    @functools.partial(
        pl.kernel,
        out_type=(
            jax.ShapeDtypeStruct((n_rows, _IN_DIM), jnp.float32),
            jax.ShapeDtypeStruct((n_rows, _IN_DIM), jnp.float32),
            jax.ShapeDtypeStruct((n_rows, _IN_DIM), jnp.float32),
        ),
        mesh=mesh,
        scratch_types=[
            pltpu.VMEM((16 * _SRC_LEN,), jnp.int32),
            pltpu.VMEM((16 * _PATH_LEN,), jnp.int32),
            pltpu.VMEM((16 * _PATH_LEN, _IN_DIM), jnp.float32),
            pltpu.VMEM((16, _IN_DIM), jnp.float32),
            pltpu.SemaphoreType.DMA,
        ],
    )
    def k(srcs_r, tgts_r, paths_r, st_r, path_r,
          src_o, tgt_o, pth_o, idx8, idx12, rows, acc, sem):
        wid = lax.axis_index("s") * 2 + lax.axis_index("c")

        def reduce_rep(idx_hbm, table, out_hbm, G, idx_v):
            base = wid * rows_per_w

            def chunk(c, carry):
                rb = base + c * 16
                pltpu.sync_copy(idx_hbm.at[pl.ds(rb * G, 16 * G)], idx_v)
                pltpu.async_copy(table.at[idx_v],
                                 rows.at[pl.ds(0, 16 * G)], sem).wait()

                def rowfn(r, carry2):
                    for v in range(_IN_DIM // 16):
                        s = rows[r * G, pl.ds(v * 16, 16)]
                        for j in range(1, G):
                            s = s + rows[r * G + j, pl.ds(v * 16, 16)]
                        acc[r, pl.ds(v * 16, 16)] = s
                    return carry2

                lax.fori_loop(0, 16, rowfn, 0)
                pltpu.sync_copy(acc, out_hbm.at[pl.ds(rb, 16)])
                return carry

            lax.fori_loop(0, rows_per_w // 16, chunk, 0)

        reduce_rep(srcs_r, st_r, src_o, _SRC_LEN, idx8)
        reduce_rep(tgts_r, st_r, tgt_o, _SRC_LEN, idx8)
        reduce_rep(paths_r, path_r, pth_o, _PATH_LEN, idx12)

    return k(srcs_i, tgts_i, paths_i, st_t, path_t)


def _fuse_body(ss, ts, ps, stok, spos, ttok, tpos, ptok, ppos,
               r0st, r0p, pose, wp, bp, wm, bm, csum, dlens, mixed_o, h_o):
    n = _B * _P
    pos_tab = pose[...].astype(jnp.bfloat16)

    def rep(sum_ref, tok_ref, pos_ref, row0_ref, G):
        tok = tok_ref[...]
        posi = pos_ref[...]
        mask = (tok != 0).astype(jnp.float32)
        cnt = mask.sum(axis=1, keepdims=True)
        cnt0 = G - cnt
        iota32 = lax.broadcasted_iota(jnp.int32, (n, 32), 1)
        hist = jnp.zeros((n, 32), jnp.float32)
        for j in range(G):
            hist = hist + (posi[:, j:j + 1] == iota32).astype(jnp.float32) * mask[:, j:j + 1]
        pos_contrib = jnp.dot(hist.astype(jnp.bfloat16), pos_tab,
                              preferred_element_type=jnp.float32)
        return (sum_ref[...] - cnt0 * row0_ref[...] + pos_contrib) / jnp.maximum(cnt, 1.0)

    sr = rep(ss, stok, spos, r0st, _SRC_LEN)
    tr = rep(ts, ttok, tpos, r0st, _SRC_LEN)
    pr = rep(ps, ptok, ppos, r0p, _PATH_LEN)

    wp_v = wp[...].astype(jnp.bfloat16)
    ep = jnp.tanh(
        jnp.dot(sr.astype(jnp.bfloat16), wp_v[0:128], preferred_element_type=jnp.float32)
        + jnp.dot(tr.astype(jnp.bfloat16), wp_v[128:256], preferred_element_type=jnp.float32)
        + jnp.dot(pr.astype(jnp.bfloat16), wp_v[256:384], preferred_element_type=jnp.float32)
        + bp[...])
    mixed_o[...] = jnp.tanh(
        jnp.dot(ep.astype(jnp.bfloat16), wm[...].astype(jnp.bfloat16),
                preferred_element_type=jnp.float32) + bm[...])
    gids = lax.broadcasted_iota(jnp.int32, (_B, n), 1) // _P
    sel = (gids == lax.broadcasted_iota(jnp.int32, (_B, n), 0)).astype(jnp.float32)
    psum = jnp.dot(sel, ep, preferred_element_type=jnp.float32)
    hf = (psum + csum[...]) / dlens[...]
    h_o[...] = jnp.broadcast_to(hf[None], (_NUM_LAYERS, _B, _H_DIM))


def _tc_fuse(src_sum, tgt_sum, pth_sum, stok, spos, ttok, tpos, ptok, ppos,
             r0st, r0p, pos_emb, wp, bp, wm, bm, csum, dlens):
    n = _B * _P
    return pl.pallas_call(
        _fuse_body,
        out_shape=(
            jax.ShapeDtypeStruct((n, _H_DIM), jnp.float32),
            jax.ShapeDtypeStruct((_NUM_LAYERS, _B, _H_DIM), jnp.float32),
        ),
    )(src_sum, tgt_sum, pth_sum, stok, spos, ttok, tpos, ptok, ppos,
      r0st, r0p, pos_emb, wp, bp, wm, bm, csum, dlens)


_TCHUNK = 128


def _gru_body(ctx_ref, wx_ref, wh_ref, bg_ref, len_ref,
              enc_ref, csum_o, h_s, csum_s, xg_s, ys_s):
    i = pl.program_id(0)

    @pl.when(i == 0)
    def _init():
        h_s[...] = jnp.zeros_like(h_s)
        csum_s[...] = jnp.zeros_like(csum_s)

    x = ctx_ref[...].astype(jnp.bfloat16)
    xg_s[...] = jnp.dot(x, wx_ref[...], preferred_element_type=jnp.float32) + bg_ref[...]
    lens = len_ref[...]
    wh = wh_ref[...]
    _G = _B // 2
    _K = _H_DIM // 2

    def hdot(h):
        hb16 = h.astype(jnp.bfloat16)
        return (jnp.dot(hb16[:, :_K], wh[:_K], preferred_element_type=jnp.float32)
                + jnp.dot(hb16[:, _K:], wh[_K:], preferred_element_type=jnp.float32))

    def sig(x):
        return 0.5 * jnp.tanh(0.5 * x) + 0.5

    def gates(xgt, hg, h, t, lens_g):
        xr = xgt[:, 0:_H_DIM]
        xz = xgt[:, _H_DIM:2 * _H_DIM]
        xn = xgt[:, 2 * _H_DIM:]
        hr = hg[:, 0:_H_DIM]
        hz = hg[:, _H_DIM:2 * _H_DIM]
        hn = hg[:, 2 * _H_DIM:]
        r = sig(xr + hr)
        z = sig(xz + hz)
        nn_ = jnp.tanh(xn + r * hn)
        h_new = (1.0 - z) * nn_ + z * h
        m = (i * _TCHUNK + t) < lens_g
        h_out = jnp.where(m, h_new, h)
        return h_out, m

    def step(t, carry):
        h_a, h_b, hg_b = carry
        hg_a = hdot(h_a)
        xgt_b = xg_s[pl.ds(t * _B + _G, _G), :]
        hb_out, mb = gates(xgt_b, hg_b, h_b, t, lens[_G:])
        ys_s[pl.ds(t * _B + _G, _G), :] = hb_out
        csum_s[_G:, :] = csum_s[_G:, :] + jnp.where(mb, hb_out, 0.0)
        hg_b_next = hdot(hb_out)
        xgt_a = xg_s[pl.ds(t * _B, _G), :]
        ha_out, ma = gates(xgt_a, hg_a, h_a, t, lens[0:_G])
        ys_s[pl.ds(t * _B, _G), :] = ha_out
        csum_s[0:_G, :] = csum_s[0:_G, :] + jnp.where(ma, ha_out, 0.0)
        return ha_out, hb_out, hg_b_next

    h0 = h_s[...]
    hg_b0 = hdot(h0[_G:])
    ha_f, hb_f, _ = lax.fori_loop(0, _TCHUNK, step, (h0[0:_G], h0[_G:], hg_b0))
    h_s[...] = jnp.concatenate([ha_f, hb_f], axis=0)
    enc_ref[...] = ys_s[...].reshape(_TCHUNK, _B, _H_DIM).transpose(1, 0, 2)

    @pl.when(i == (_T // _TCHUNK) - 1)
    def _fin():
        csum_o[...] = csum_s[...]


def _tc_gru(ctx_in_tm, wx, wh, bg, lens):
    nchunks = _T // _TCHUNK
    return pl.pallas_call(
        _gru_body,
        grid=(nchunks,),
        in_specs=[
            pl.BlockSpec((_TCHUNK * _B, _IN_DIM), lambda i: (i, 0)),
            pl.BlockSpec((_IN_DIM, 3 * _H_DIM), lambda i: (0, 0)),
            pl.BlockSpec((_H_DIM, 3 * _H_DIM), lambda i: (0, 0)),
            pl.BlockSpec((1, 3 * _H_DIM), lambda i: (0, 0)),
            pl.BlockSpec((_B, 1), lambda i: (0, 0)),
        ],
        out_specs=[
            pl.BlockSpec((_B, _TCHUNK, _H_DIM), lambda i: (0, i, 0)),
            pl.BlockSpec((_B, _H_DIM), lambda i: (0, 0)),
        ],
        out_shape=(
            jax.ShapeDtypeStruct((_B, _T, _H_DIM), jnp.float32),
            jax.ShapeDtypeStruct((_B, _H_DIM), jnp.float32),
        ),
        scratch_shapes=[
            pltpu.VMEM((_B, _H_DIM), jnp.float32),
            pltpu.VMEM((_B, _H_DIM), jnp.float32),
            pltpu.VMEM((_TCHUNK * _B, 3 * _H_DIM), jnp.float32),
            pltpu.VMEM((_TCHUNK * _B, _H_DIM), jnp.float32),
        ],
    )(ctx_in_tm, wx, wh, bg, lens)


def kernel(packed_srcs, packed_srcs_positions, packed_tgts, packed_tgts_positions,
           packed_paths, packed_paths_positions, packed_ctx, ctx_lengths,
           focus_num_of_paths, path_emb, src_tgt_emb, pos_emb, ctx_table,
           W_path, b_path, W_mix, b_mix, Wx, Wh, bg):
    ctx_rows_tm = _sc_ctx_gather(packed_ctx.T.reshape(-1), ctx_table)

    src_sum, tgt_sum, pth_sum = _sc_path_sums(
        packed_srcs.reshape(-1), packed_tgts.reshape(-1),
        packed_paths.reshape(-1), src_tgt_emb, path_emb)

    lens = ctx_lengths.reshape(_B, 1)
    enc, csum = _tc_gru(ctx_rows_tm, Wx.astype(jnp.bfloat16),
                        Wh.astype(jnp.bfloat16), bg.reshape(1, -1), lens)

    dlens = (focus_num_of_paths + ctx_lengths).astype(jnp.float32).reshape(_B, 1)
    mixed, h = _tc_fuse(
        src_sum, tgt_sum, pth_sum,
        packed_srcs, packed_srcs_positions,
        packed_tgts, packed_tgts_positions,
        packed_paths, packed_paths_positions,
        src_tgt_emb[0:1], path_emb[0:1], pos_emb,
        W_path, b_path.reshape(1, -1), W_mix, b_mix.reshape(1, -1),
        csum, dlens)

    return (mixed.reshape(_B, _P, _H_DIM), enc, h)


# speedup vs baseline: 1.1769x; 1.1452x over previous
import functools

import jax
import jax.numpy as jnp
from jax import lax
from jax.experimental import pallas as pl
from jax.experimental.pallas import tpu as pltpu
from jax.experimental.pallas import tpu_sc as plsc

_B = 16
_P = 128
_T = 512
_SRC_LEN = 8
_PATH_LEN = 12
_IN_DIM = 128
_H_DIM = 256
_NUM_LAYERS = 2
_NW = 32


def _sc_ctx_gather(ctx_i, ctx_t):
    ctx_per_w = (_B * _T) // _NW
    mesh = plsc.VectorSubcoreMesh(core_axis_name="c", subcore_axis_name="s")

    @functools.partial(
        pl.kernel,
        out_type=jax.ShapeDtypeStruct((_B * _T, _IN_DIM), jnp.float32),
        mesh=mesh,
        scratch_types=[
            pltpu.VMEM((ctx_per_w,), jnp.int32),
            pltpu.VMEM((ctx_per_w, _IN_DIM), jnp.float32),
            pltpu.SemaphoreType.DMA,
        ],
    )
    def k(ctxi_r, ctxt_r, ctx_o, idxc, rows, sem):
        wid = lax.axis_index("s") * 2 + lax.axis_index("c")
        cb = wid * ctx_per_w
        pltpu.sync_copy(ctxi_r.at[pl.ds(cb, ctx_per_w)], idxc)
        pltpu.async_copy(ctxt_r.at[idxc], rows, sem).wait()
        pltpu.sync_copy(rows, ctx_o.at[pl.ds(cb, ctx_per_w)])

    return k(ctx_i, ctx_t)


def _sc_path_sums(srcs_i, tgts_i, paths_i, st_t, path_t):
    n_rows = _B * _P
    rows_per_w = n_rows // _NW
    mesh = plsc.VectorSubcoreMesh(core_axis_name="c", subcore_axis_name="s")

    @functools.partial(
        pl.kernel,
        out_type=(
            jax.ShapeDtypeStruct((n_rows, _IN_DIM), jnp.float32),
            jax.ShapeDtypeStruct((n_rows, _IN_DIM), jnp.float32),
            jax.ShapeDtypeStruct((n_rows, _IN_DIM), jnp.float32),
        ),
        mesh=mesh,
        scratch_types=[
            pltpu.VMEM((16 * _SRC_LEN,), jnp.int32),
            pltpu.VMEM((16 * _PATH_LEN,), jnp.int32),
            pltpu.VMEM((16 * _PATH_LEN, _IN_DIM), jnp.float32),
            pltpu.VMEM((16, _IN_DIM), jnp.float32),
            pltpu.SemaphoreType.DMA,
        ],
    )
    def k(srcs_r, tgts_r, paths_r, st_r, path_r,
          src_o, tgt_o, pth_o, idx8, idx12, rows, acc, sem):
        wid = lax.axis_index("s") * 2 + lax.axis_index("c")

        def reduce_rep(idx_hbm, table, out_hbm, G, idx_v):
            base = wid * rows_per_w

            def chunk(c, carry):
                rb = base + c * 16
                pltpu.sync_copy(idx_hbm.at[pl.ds(rb * G, 16 * G)], idx_v)
                pltpu.async_copy(table.at[idx_v],
                                 rows.at[pl.ds(0, 16 * G)], sem).wait()

                def rowfn(r, carry2):
                    for v in range(_IN_DIM // 16):
                        s = rows[r * G, pl.ds(v * 16, 16)]
                        for j in range(1, G):
                            s = s + rows[r * G + j, pl.ds(v * 16, 16)]
                        acc[r, pl.ds(v * 16, 16)] = s
                    return carry2

                lax.fori_loop(0, 16, rowfn, 0)
                pltpu.sync_copy(acc, out_hbm.at[pl.ds(rb, 16)])
                return carry

            lax.fori_loop(0, rows_per_w // 16, chunk, 0)

        reduce_rep(srcs_r, st_r, src_o, _SRC_LEN, idx8)
        reduce_rep(tgts_r, st_r, tgt_o, _SRC_LEN, idx8)
        reduce_rep(paths_r, path_r, pth_o, _PATH_LEN, idx12)

    return k(srcs_i, tgts_i, paths_i, st_t, path_t)


def _fuse_body(ss, ts, ps, stok, spos, ttok, tpos, ptok, ppos,
               r0st, r0p, pose, wp, bp, wm, bm, csum, dlens, mixed_o, h_o):
    n = _B * _P
    pos_tab = pose[...].astype(jnp.bfloat16)

    def rep(sum_ref, tok_ref, pos_ref, row0_ref, G):
        tok = tok_ref[...]
        posi = pos_ref[...]
        mask = (tok != 0).astype(jnp.float32)
        cnt = mask.sum(axis=1, keepdims=True)
        cnt0 = G - cnt
        iota32 = lax.broadcasted_iota(jnp.int32, (n, 32), 1)
        hist = jnp.zeros((n, 32), jnp.float32)
        for j in range(G):
            hist = hist + (posi[:, j:j + 1] == iota32).astype(jnp.float32) * mask[:, j:j + 1]
        pos_contrib = jnp.dot(hist.astype(jnp.bfloat16), pos_tab,
                              preferred_element_type=jnp.float32)
        return (sum_ref[...] - cnt0 * row0_ref[...] + pos_contrib) / jnp.maximum(cnt, 1.0)

    sr = rep(ss, stok, spos, r0st, _SRC_LEN)
    tr = rep(ts, ttok, tpos, r0st, _SRC_LEN)
    pr = rep(ps, ptok, ppos, r0p, _PATH_LEN)

    wp_v = wp[...].astype(jnp.bfloat16)
    ep = jnp.tanh(
        jnp.dot(sr.astype(jnp.bfloat16), wp_v[0:128], preferred_element_type=jnp.float32)
        + jnp.dot(tr.astype(jnp.bfloat16), wp_v[128:256], preferred_element_type=jnp.float32)
        + jnp.dot(pr.astype(jnp.bfloat16), wp_v[256:384], preferred_element_type=jnp.float32)
        + bp[...])
    mixed_o[...] = jnp.tanh(
        jnp.dot(ep.astype(jnp.bfloat16), wm[...].astype(jnp.bfloat16),
                preferred_element_type=jnp.float32) + bm[...])
    gids = lax.broadcasted_iota(jnp.int32, (_B, n), 1) // _P
    sel = (gids == lax.broadcasted_iota(jnp.int32, (_B, n), 0)).astype(jnp.float32)
    psum = jnp.dot(sel, ep, preferred_element_type=jnp.float32)
    hf = (psum + csum[...]) / dlens[...]
    h_o[...] = jnp.broadcast_to(hf[None], (_NUM_LAYERS, _B, _H_DIM))


def _tc_fuse(src_sum, tgt_sum, pth_sum, stok, spos, ttok, tpos, ptok, ppos,
             r0st, r0p, pos_emb, wp, bp, wm, bm, csum, dlens):
    n = _B * _P
    return pl.pallas_call(
        _fuse_body,
        out_shape=(
            jax.ShapeDtypeStruct((n, _H_DIM), jnp.float32),
            jax.ShapeDtypeStruct((_NUM_LAYERS, _B, _H_DIM), jnp.float32),
        ),
    )(src_sum, tgt_sum, pth_sum, stok, spos, ttok, tpos, ptok, ppos,
      r0st, r0p, pos_emb, wp, bp, wm, bm, csum, dlens)


_TCHUNK = 64


def _gru_body(ctx_ref, wx_ref, wh_ref, bg_ref, len_ref,
              enc_ref, csum_o, h_s, csum_s, xg_s, ys_s):
    i = pl.program_id(0)

    @pl.when(i == 0)
    def _init():
        h_s[...] = jnp.zeros_like(h_s)
        csum_s[...] = jnp.zeros_like(csum_s)

    x = ctx_ref[...].astype(jnp.bfloat16)
    xg_s[...] = jnp.dot(x, wx_ref[...], preferred_element_type=jnp.float32) + bg_ref[...]
    lens = len_ref[...]
    wh = wh_ref[...]
    _G = _B // 2
    def hdot(h):
        return jnp.dot(h.astype(jnp.bfloat16), wh, preferred_element_type=jnp.float32)

    sig = jax.nn.sigmoid

    def gates(xgt, hg, h, t, lens_g):
        xr = xgt[:, 0:_H_DIM]
        xz = xgt[:, _H_DIM:2 * _H_DIM]
        xn = xgt[:, 2 * _H_DIM:]
        hr = hg[:, 0:_H_DIM]
        hz = hg[:, _H_DIM:2 * _H_DIM]
        hn = hg[:, 2 * _H_DIM:]
        r = sig(xr + hr)
        z = sig(xz + hz)
        nn_ = jnp.tanh(xn + r * hn)
        h_new = (1.0 - z) * nn_ + z * h
        m = (i * _TCHUNK + t) < lens_g
        h_out = jnp.where(m, h_new, h)
        return h_out, m

    def step(t, carry):
        h_a, h_b, hg_b = carry
        hg_a = hdot(h_a)
        xgt_b = xg_s[pl.ds(t * _B + _G, _G), :]
        hb_out, mb = gates(xgt_b, hg_b, h_b, t, lens[_G:])
        ys_s[pl.ds(t * _B + _G, _G), :] = hb_out
        csum_s[_G:, :] = csum_s[_G:, :] + jnp.where(mb, hb_out, 0.0)
        hg_b_next = hdot(hb_out)
        xgt_a = xg_s[pl.ds(t * _B, _G), :]
        ha_out, ma = gates(xgt_a, hg_a, h_a, t, lens[0:_G])
        ys_s[pl.ds(t * _B, _G), :] = ha_out
        csum_s[0:_G, :] = csum_s[0:_G, :] + jnp.where(ma, ha_out, 0.0)
        return ha_out, hb_out, hg_b_next

    def step2(u, carry):
        carry = step(2 * u, carry)
        return step(2 * u + 1, carry)

    h0 = h_s[...]
    hg_b0 = hdot(h0[_G:])
    ha_f, hb_f, _ = lax.fori_loop(0, _TCHUNK // 2, step2, (h0[0:_G], h0[_G:], hg_b0))
    h_s[...] = jnp.concatenate([ha_f, hb_f], axis=0)
    enc_ref[...] = ys_s[...].reshape(_TCHUNK, _B, _H_DIM).transpose(1, 0, 2)

    @pl.when(i == (_T // _TCHUNK) - 1)
    def _fin():
        csum_o[...] = csum_s[...]


def _tc_gru(ctx_in_tm, wx, wh, bg, lens):
    nchunks = _T // _TCHUNK
    return pl.pallas_call(
        _gru_body,
        grid=(nchunks,),
        in_specs=[
            pl.BlockSpec((_TCHUNK * _B, _IN_DIM), lambda i: (i, 0)),
            pl.BlockSpec((_IN_DIM, 3 * _H_DIM), lambda i: (0, 0)),
            pl.BlockSpec((_H_DIM, 3 * _H_DIM), lambda i: (0, 0)),
            pl.BlockSpec((1, 3 * _H_DIM), lambda i: (0, 0)),
            pl.BlockSpec((_B, 1), lambda i: (0, 0)),
        ],
        out_specs=[
            pl.BlockSpec((_B, _TCHUNK, _H_DIM), lambda i: (0, i, 0)),
            pl.BlockSpec((_B, _H_DIM), lambda i: (0, 0)),
        ],
        out_shape=(
            jax.ShapeDtypeStruct((_B, _T, _H_DIM), jnp.float32),
            jax.ShapeDtypeStruct((_B, _H_DIM), jnp.float32),
        ),
        scratch_shapes=[
            pltpu.VMEM((_B, _H_DIM), jnp.float32),
            pltpu.VMEM((_B, _H_DIM), jnp.float32),
            pltpu.VMEM((_TCHUNK * _B, 3 * _H_DIM), jnp.float32),
            pltpu.VMEM((_TCHUNK * _B, _H_DIM), jnp.float32),
        ],
    )(ctx_in_tm, wx, wh, bg, lens)


def kernel(packed_srcs, packed_srcs_positions, packed_tgts, packed_tgts_positions,
           packed_paths, packed_paths_positions, packed_ctx, ctx_lengths,
           focus_num_of_paths, path_emb, src_tgt_emb, pos_emb, ctx_table,
           W_path, b_path, W_mix, b_mix, Wx, Wh, bg):
    ctx_rows_tm = _sc_ctx_gather(packed_ctx.T.reshape(-1), ctx_table)

    src_sum, tgt_sum, pth_sum = _sc_path_sums(
        packed_srcs.reshape(-1), packed_tgts.reshape(-1),
        packed_paths.reshape(-1), src_tgt_emb, path_emb)

    lens = ctx_lengths.reshape(_B, 1)
    enc, csum = _tc_gru(ctx_rows_tm, Wx.astype(jnp.bfloat16),
                        Wh.astype(jnp.bfloat16), bg.reshape(1, -1), lens)

    dlens = (focus_num_of_paths + ctx_lengths).astype(jnp.float32).reshape(_B, 1)
    mixed, h = _tc_fuse(
        src_sum, tgt_sum, pth_sum,
        packed_srcs, packed_srcs_positions,
        packed_tgts, packed_tgts_positions,
        packed_paths, packed_paths_positions,
        src_tgt_emb[0:1], path_emb[0:1], pos_emb,
        W_path, b_path.reshape(1, -1), W_mix, b_mix.reshape(1, -1),
        csum, dlens)

    return (mixed.reshape(_B, _P, _H_DIM), enc, h)


# speedup vs baseline: 1.2444x; 1.0574x over previous
import functools

import jax
import jax.numpy as jnp
from jax import lax
from jax.experimental import pallas as pl
from jax.experimental.pallas import tpu as pltpu
from jax.experimental.pallas import tpu_sc as plsc

_B = 16
_P = 128
_T = 512
_SRC_LEN = 8
_PATH_LEN = 12
_IN_DIM = 128
_H_DIM = 256
_NUM_LAYERS = 2
_NW = 32


def _sc_ctx_gather(ctx_i, ctx_t):
    ctx_per_w = (_B * _T) // _NW
    mesh = plsc.VectorSubcoreMesh(core_axis_name="c", subcore_axis_name="s")

    @functools.partial(
        pl.kernel,
        out_type=jax.ShapeDtypeStruct((_B * _T, _IN_DIM), jnp.float32),
        mesh=mesh,
        scratch_types=[
            pltpu.VMEM((ctx_per_w,), jnp.int32),
            pltpu.VMEM((ctx_per_w, _IN_DIM), jnp.float32),
            pltpu.SemaphoreType.DMA,
        ],
    )
    def k(ctxi_r, ctxt_r, ctx_o, idxc, rows, sem):
        wid = lax.axis_index("s") * 2 + lax.axis_index("c")
        cb = wid * ctx_per_w
        pltpu.sync_copy(ctxi_r.at[pl.ds(cb, ctx_per_w)], idxc)
        pltpu.async_copy(ctxt_r.at[idxc], rows, sem).wait()
        pltpu.sync_copy(rows, ctx_o.at[pl.ds(cb, ctx_per_w)])

    return k(ctx_i, ctx_t)


def _sc_path_sums(srcs_i, tgts_i, paths_i, st_t, path_t):
    n_rows = _B * _P
    rows_per_w = n_rows // _NW
    mesh = plsc.VectorSubcoreMesh(core_axis_name="c", subcore_axis_name="s")

    @functools.partial(
        pl.kernel,
        out_type=(
            jax.ShapeDtypeStruct((n_rows, _IN_DIM), jnp.float32),
            jax.ShapeDtypeStruct((n_rows, _IN_DIM), jnp.float32),
            jax.ShapeDtypeStruct((n_rows, _IN_DIM), jnp.float32),
        ),
        mesh=mesh,
        scratch_types=[
            pltpu.VMEM((16 * _SRC_LEN,), jnp.int32),
            pltpu.VMEM((16 * _PATH_LEN,), jnp.int32),
            pltpu.VMEM((16 * _PATH_LEN, _IN_DIM), jnp.float32),
            pltpu.VMEM((16, _IN_DIM), jnp.float32),
            pltpu.SemaphoreType.DMA,
        ],
    )
    def k(srcs_r, tgts_r, paths_r, st_r, path_r,
          src_o, tgt_o, pth_o, idx8, idx12, rows, acc, sem):
        wid = lax.axis_index("s") * 2 + lax.axis_index("c")

        def reduce_rep(idx_hbm, table, out_hbm, G, idx_v):
            base = wid * rows_per_w

            def chunk(c, carry):
                rb = base + c * 16
                pltpu.sync_copy(idx_hbm.at[pl.ds(rb * G, 16 * G)], idx_v)
                pltpu.async_copy(table.at[idx_v],
                                 rows.at[pl.ds(0, 16 * G)], sem).wait()

                def rowfn(r, carry2):
                    for v in range(_IN_DIM // 16):
                        s = rows[r * G, pl.ds(v * 16, 16)]
                        for j in range(1, G):
                            s = s + rows[r * G + j, pl.ds(v * 16, 16)]
                        acc[r, pl.ds(v * 16, 16)] = s
                    return carry2

                lax.fori_loop(0, 16, rowfn, 0)
                pltpu.sync_copy(acc, out_hbm.at[pl.ds(rb, 16)])
                return carry

            lax.fori_loop(0, rows_per_w // 16, chunk, 0)

        reduce_rep(srcs_r, st_r, src_o, _SRC_LEN, idx8)
        reduce_rep(tgts_r, st_r, tgt_o, _SRC_LEN, idx8)
        reduce_rep(paths_r, path_r, pth_o, _PATH_LEN, idx12)

    return k(srcs_i, tgts_i, paths_i, st_t, path_t)


def _fuse_body(ss, ts, ps, stok, spos, ttok, tpos, ptok, ppos,
               r0st, r0p, pose, wp, bp, wm, bm, csum, dlens, mixed_o, h_o):
    n = _B * _P
    pos_tab = pose[...].astype(jnp.bfloat16)

    def rep(sum_ref, tok_ref, pos_ref, row0_ref, G):
        tok = tok_ref[...]
        posi = pos_ref[...]
        mask = (tok != 0).astype(jnp.float32)
        cnt = mask.sum(axis=1, keepdims=True)
        cnt0 = G - cnt
        iota32 = lax.broadcasted_iota(jnp.int32, (n, 32), 1)
        hist = jnp.zeros((n, 32), jnp.float32)
        for j in range(G):
            hist = hist + (posi[:, j:j + 1] == iota32).astype(jnp.float32) * mask[:, j:j + 1]
        pos_contrib = jnp.dot(hist.astype(jnp.bfloat16), pos_tab,
                              preferred_element_type=jnp.float32)
        return (sum_ref[...] - cnt0 * row0_ref[...] + pos_contrib) / jnp.maximum(cnt, 1.0)

    sr = rep(ss, stok, spos, r0st, _SRC_LEN)
    tr = rep(ts, ttok, tpos, r0st, _SRC_LEN)
    pr = rep(ps, ptok, ppos, r0p, _PATH_LEN)

    wp_v = wp[...].astype(jnp.bfloat16)
    ep = jnp.tanh(
        jnp.dot(sr.astype(jnp.bfloat16), wp_v[0:128], preferred_element_type=jnp.float32)
        + jnp.dot(tr.astype(jnp.bfloat16), wp_v[128:256], preferred_element_type=jnp.float32)
        + jnp.dot(pr.astype(jnp.bfloat16), wp_v[256:384], preferred_element_type=jnp.float32)
        + bp[...])
    mixed_o[...] = jnp.tanh(
        jnp.dot(ep.astype(jnp.bfloat16), wm[...].astype(jnp.bfloat16),
                preferred_element_type=jnp.float32) + bm[...])
    gids = lax.broadcasted_iota(jnp.int32, (_B, n), 1) // _P
    sel = (gids == lax.broadcasted_iota(jnp.int32, (_B, n), 0)).astype(jnp.float32)
    psum = jnp.dot(sel, ep, preferred_element_type=jnp.float32)
    hf = (psum + csum[...]) / dlens[...]
    h_o[...] = jnp.broadcast_to(hf[None], (_NUM_LAYERS, _B, _H_DIM))


def _tc_fuse(src_sum, tgt_sum, pth_sum, stok, spos, ttok, tpos, ptok, ppos,
             r0st, r0p, pos_emb, wp, bp, wm, bm, csum, dlens):
    n = _B * _P
    return pl.pallas_call(
        _fuse_body,
        out_shape=(
            jax.ShapeDtypeStruct((n, _H_DIM), jnp.float32),
            jax.ShapeDtypeStruct((_NUM_LAYERS, _B, _H_DIM), jnp.float32),
        ),
    )(src_sum, tgt_sum, pth_sum, stok, spos, ttok, tpos, ptok, ppos,
      r0st, r0p, pos_emb, wp, bp, wm, bm, csum, dlens)


_TCHUNK = 64


def _gru_body(ctx_ref, wx_ref, wh_ref, bg_ref, len_ref,
              enc_ref, csum_o, h_s, csum_s, xg_s, ys_s):
    i = pl.program_id(0)

    @pl.when(i == 0)
    def _init():
        h_s[...] = jnp.zeros_like(h_s)
        csum_s[...] = jnp.zeros_like(csum_s)

    x = ctx_ref[...].astype(jnp.bfloat16)
    xg_s[...] = jnp.dot(x, wx_ref[...], preferred_element_type=jnp.float32) + bg_ref[...]
    lens = len_ref[...]
    wh = wh_ref[...]
    _G = _B // 2
    def hdot(h):
        return jnp.dot(h.astype(jnp.bfloat16), wh, preferred_element_type=jnp.float32)

    sig = jax.nn.sigmoid

    def gates(xgt, hg, h, t, lens_g):
        xr = xgt[:, 0:_H_DIM]
        xz = xgt[:, _H_DIM:2 * _H_DIM]
        xn = xgt[:, 2 * _H_DIM:]
        hr = hg[:, 0:_H_DIM]
        hz = hg[:, _H_DIM:2 * _H_DIM]
        hn = hg[:, 2 * _H_DIM:]
        r = sig(xr + hr)
        z = sig(xz + hz)
        nn_ = jnp.tanh(xn + r * hn)
        h_new = (1.0 - z) * nn_ + z * h
        m = (i * _TCHUNK + t) < lens_g
        h_out = jnp.where(m, h_new, h)
        return h_out, m

    def step(t, carry):
        h_a, h_b, hg_b = carry
        hg_a = hdot(h_a)
        xgt_b = xg_s[pl.ds(t * _B + _G, _G), :]
        hb_out, mb = gates(xgt_b, hg_b, h_b, t, lens[_G:])
        ys_s[pl.ds(t * _B + _G, _G), :] = hb_out
        csum_s[_G:, :] = csum_s[_G:, :] + jnp.where(mb, hb_out, 0.0)
        hg_b_next = hdot(hb_out)
        xgt_a = xg_s[pl.ds(t * _B, _G), :]
        ha_out, ma = gates(xgt_a, hg_a, h_a, t, lens[0:_G])
        ys_s[pl.ds(t * _B, _G), :] = ha_out
        csum_s[0:_G, :] = csum_s[0:_G, :] + jnp.where(ma, ha_out, 0.0)
        return ha_out, hb_out, hg_b_next

    _UNROLL = 4

    def stepu(u, carry):
        for k in range(_UNROLL):
            carry = step(_UNROLL * u + k, carry)
        return carry

    h0 = h_s[...]
    hg_b0 = hdot(h0[_G:])
    ha_f, hb_f, _ = lax.fori_loop(0, _TCHUNK // _UNROLL, stepu,
                                  (h0[0:_G], h0[_G:], hg_b0))
    h_s[...] = jnp.concatenate([ha_f, hb_f], axis=0)
    enc_ref[...] = ys_s[...].reshape(_TCHUNK, _B, _H_DIM).transpose(1, 0, 2)

    @pl.when(i == (_T // _TCHUNK) - 1)
    def _fin():
        csum_o[...] = csum_s[...]


def _tc_gru(ctx_in_tm, wx, wh, bg, lens):
    nchunks = _T // _TCHUNK
    return pl.pallas_call(
        _gru_body,
        grid=(nchunks,),
        in_specs=[
            pl.BlockSpec((_TCHUNK * _B, _IN_DIM), lambda i: (i, 0)),
            pl.BlockSpec((_IN_DIM, 3 * _H_DIM), lambda i: (0, 0)),
            pl.BlockSpec((_H_DIM, 3 * _H_DIM), lambda i: (0, 0)),
            pl.BlockSpec((1, 3 * _H_DIM), lambda i: (0, 0)),
            pl.BlockSpec((_B, 1), lambda i: (0, 0)),
        ],
        out_specs=[
            pl.BlockSpec((_B, _TCHUNK, _H_DIM), lambda i: (0, i, 0)),
            pl.BlockSpec((_B, _H_DIM), lambda i: (0, 0)),
        ],
        out_shape=(
            jax.ShapeDtypeStruct((_B, _T, _H_DIM), jnp.float32),
            jax.ShapeDtypeStruct((_B, _H_DIM), jnp.float32),
        ),
        scratch_shapes=[
            pltpu.VMEM((_B, _H_DIM), jnp.float32),
            pltpu.VMEM((_B, _H_DIM), jnp.float32),
            pltpu.VMEM((_TCHUNK * _B, 3 * _H_DIM), jnp.float32),
            pltpu.VMEM((_TCHUNK * _B, _H_DIM), jnp.float32),
        ],
    )(ctx_in_tm, wx, wh, bg, lens)


def kernel(packed_srcs, packed_srcs_positions, packed_tgts, packed_tgts_positions,
           packed_paths, packed_paths_positions, packed_ctx, ctx_lengths,
           focus_num_of_paths, path_emb, src_tgt_emb, pos_emb, ctx_table,
           W_path, b_path, W_mix, b_mix, Wx, Wh, bg):
    ctx_rows_tm = _sc_ctx_gather(packed_ctx.T.reshape(-1), ctx_table)

    src_sum, tgt_sum, pth_sum = _sc_path_sums(
        packed_srcs.reshape(-1), packed_tgts.reshape(-1),
        packed_paths.reshape(-1), src_tgt_emb, path_emb)

    lens = ctx_lengths.reshape(_B, 1)
    enc, csum = _tc_gru(ctx_rows_tm, Wx.astype(jnp.bfloat16),
                        Wh.astype(jnp.bfloat16), bg.reshape(1, -1), lens)

    dlens = (focus_num_of_paths + ctx_lengths).astype(jnp.float32).reshape(_B, 1)
    mixed, h = _tc_fuse(
        src_sum, tgt_sum, pth_sum,
        packed_srcs, packed_srcs_positions,
        packed_tgts, packed_tgts_positions,
        packed_paths, packed_paths_positions,
        src_tgt_emb[0:1], path_emb[0:1], pos_emb,
        W_path, b_path.reshape(1, -1), W_mix, b_mix.reshape(1, -1),
        csum, dlens)

    return (mixed.reshape(_B, _P, _H_DIM), enc, h)


# speedup vs baseline: 1.2783x; 1.0273x over previous
import functools

import jax
import jax.numpy as jnp
from jax import lax
from jax.experimental import pallas as pl
from jax.experimental.pallas import tpu as pltpu
from jax.experimental.pallas import tpu_sc as plsc

_B = 16
_P = 128
_T = 512
_SRC_LEN = 8
_PATH_LEN = 12
_IN_DIM = 128
_H_DIM = 256
_NUM_LAYERS = 2
_NW = 32


def _sc_ctx_gather(ctx_i, ctx_t):
    ctx_per_w = (_B * _T) // _NW
    mesh = plsc.VectorSubcoreMesh(core_axis_name="c", subcore_axis_name="s")

    @functools.partial(
        pl.kernel,
        out_type=jax.ShapeDtypeStruct((_B * _T, _IN_DIM), jnp.float32),
        mesh=mesh,
        scratch_types=[
            pltpu.VMEM((ctx_per_w,), jnp.int32),
            pltpu.VMEM((ctx_per_w, _IN_DIM), jnp.float32),
            pltpu.SemaphoreType.DMA,
        ],
    )
    def k(ctxi_r, ctxt_r, ctx_o, idxc, rows, sem):
        wid = lax.axis_index("s") * 2 + lax.axis_index("c")
        cb = wid * ctx_per_w
        pltpu.sync_copy(ctxi_r.at[pl.ds(cb, ctx_per_w)], idxc)
        pltpu.async_copy(ctxt_r.at[idxc], rows, sem).wait()
        pltpu.sync_copy(rows, ctx_o.at[pl.ds(cb, ctx_per_w)])

    return k(ctx_i, ctx_t)


def _sc_path_sums(srcs_i, tgts_i, paths_i, st_t, path_t):
    n_rows = _B * _P
    rows_per_w = n_rows // _NW
    mesh = plsc.VectorSubcoreMesh(core_axis_name="c", subcore_axis_name="s")

    @functools.partial(
        pl.kernel,
        out_type=(
            jax.ShapeDtypeStruct((n_rows, _IN_DIM), jnp.float32),
            jax.ShapeDtypeStruct((n_rows, _IN_DIM), jnp.float32),
            jax.ShapeDtypeStruct((n_rows, _IN_DIM), jnp.float32),
        ),
        mesh=mesh,
        scratch_types=[
            pltpu.VMEM((16 * _SRC_LEN,), jnp.int32),
            pltpu.VMEM((16 * _PATH_LEN,), jnp.int32),
            pltpu.VMEM((16 * _PATH_LEN, _IN_DIM), jnp.float32),
            pltpu.VMEM((16, _IN_DIM), jnp.float32),
            pltpu.SemaphoreType.DMA,
        ],
    )
    def k(srcs_r, tgts_r, paths_r, st_r, path_r,
          src_o, tgt_o, pth_o, idx8, idx12, rows, acc, sem):
        wid = lax.axis_index("s") * 2 + lax.axis_index("c")

        def reduce_rep(idx_hbm, table, out_hbm, G, idx_v):
            base = wid * rows_per_w

            def chunk(c, carry):
                rb = base + c * 16
                pltpu.sync_copy(idx_hbm.at[pl.ds(rb * G, 16 * G)], idx_v)
                pltpu.async_copy(table.at[idx_v],
                                 rows.at[pl.ds(0, 16 * G)], sem).wait()

                def rowfn(r, carry2):
                    for v in range(_IN_DIM // 16):
                        s = rows[r * G, pl.ds(v * 16, 16)]
                        for j in range(1, G):
                            s = s + rows[r * G + j, pl.ds(v * 16, 16)]
                        acc[r, pl.ds(v * 16, 16)] = s
                    return carry2

                lax.fori_loop(0, 16, rowfn, 0)
                pltpu.sync_copy(acc, out_hbm.at[pl.ds(rb, 16)])
                return carry

            lax.fori_loop(0, rows_per_w // 16, chunk, 0)

        reduce_rep(srcs_r, st_r, src_o, _SRC_LEN, idx8)
        reduce_rep(tgts_r, st_r, tgt_o, _SRC_LEN, idx8)
        reduce_rep(paths_r, path_r, pth_o, _PATH_LEN, idx12)

    return k(srcs_i, tgts_i, paths_i, st_t, path_t)


def _fuse_body(ss, ts, ps, stok, spos, ttok, tpos, ptok, ppos,
               r0st, r0p, pose, wp, bp, wm, bm, csum, dlens, mixed_o, h_o):
    n = _B * _P
    pos_tab = pose[...].astype(jnp.bfloat16)

    def rep(sum_ref, tok_ref, pos_ref, row0_ref, G):
        tok = tok_ref[...]
        posi = pos_ref[...]
        mask = (tok != 0).astype(jnp.float32)
        cnt = mask.sum(axis=1, keepdims=True)
        cnt0 = G - cnt
        iota32 = lax.broadcasted_iota(jnp.int32, (n, 32), 1)
        hist = jnp.zeros((n, 32), jnp.float32)
        for j in range(G):
            hist = hist + (posi[:, j:j + 1] == iota32).astype(jnp.float32) * mask[:, j:j + 1]
        pos_contrib = jnp.dot(hist.astype(jnp.bfloat16), pos_tab,
                              preferred_element_type=jnp.float32)
        return (sum_ref[...] - cnt0 * row0_ref[...] + pos_contrib) / jnp.maximum(cnt, 1.0)

    sr = rep(ss, stok, spos, r0st, _SRC_LEN)
    tr = rep(ts, ttok, tpos, r0st, _SRC_LEN)
    pr = rep(ps, ptok, ppos, r0p, _PATH_LEN)

    wp_v = wp[...].astype(jnp.bfloat16)
    ep = jnp.tanh(
        jnp.dot(sr.astype(jnp.bfloat16), wp_v[0:128], preferred_element_type=jnp.float32)
        + jnp.dot(tr.astype(jnp.bfloat16), wp_v[128:256], preferred_element_type=jnp.float32)
        + jnp.dot(pr.astype(jnp.bfloat16), wp_v[256:384], preferred_element_type=jnp.float32)
        + bp[...])
    mixed_o[...] = jnp.tanh(
        jnp.dot(ep.astype(jnp.bfloat16), wm[...].astype(jnp.bfloat16),
                preferred_element_type=jnp.float32) + bm[...])
    gids = lax.broadcasted_iota(jnp.int32, (_B, n), 1) // _P
    sel = (gids == lax.broadcasted_iota(jnp.int32, (_B, n), 0)).astype(jnp.float32)
    psum = jnp.dot(sel, ep, preferred_element_type=jnp.float32)
    hf = (psum + csum[...]) / dlens[...]
    h_o[...] = jnp.broadcast_to(hf[None], (_NUM_LAYERS, _B, _H_DIM))


def _tc_fuse(src_sum, tgt_sum, pth_sum, stok, spos, ttok, tpos, ptok, ppos,
             r0st, r0p, pos_emb, wp, bp, wm, bm, csum, dlens):
    n = _B * _P
    return pl.pallas_call(
        _fuse_body,
        out_shape=(
            jax.ShapeDtypeStruct((n, _H_DIM), jnp.float32),
            jax.ShapeDtypeStruct((_NUM_LAYERS, _B, _H_DIM), jnp.float32),
        ),
    )(src_sum, tgt_sum, pth_sum, stok, spos, ttok, tpos, ptok, ppos,
      r0st, r0p, pos_emb, wp, bp, wm, bm, csum, dlens)


_TCHUNK = 64


def _gru_body(ctx_ref, wx_ref, wh_ref, bg_ref, len_ref,
              enc_ref, csum_o, h_s, csum_s, xg_s, ys_s):
    i = pl.program_id(0)

    @pl.when(i == 0)
    def _init():
        h_s[...] = jnp.zeros_like(h_s)
        csum_s[...] = jnp.zeros_like(csum_s)

    x = ctx_ref[...].astype(jnp.bfloat16)
    xg_s[...] = jnp.dot(x, wx_ref[...], preferred_element_type=jnp.float32) + bg_ref[...]
    lens = len_ref[...]
    wh = wh_ref[...]
    _G = _B // 2
    def hdot(h):
        return jnp.dot(h.astype(jnp.bfloat16), wh, preferred_element_type=jnp.float32)

    sig = jax.nn.sigmoid

    def gates(xgt, hg, h, t, lens_g):
        xr = xgt[:, 0:_H_DIM]
        xz = xgt[:, _H_DIM:2 * _H_DIM]
        xn = xgt[:, 2 * _H_DIM:]
        hr = hg[:, 0:_H_DIM]
        hz = hg[:, _H_DIM:2 * _H_DIM]
        hn = hg[:, 2 * _H_DIM:]
        r = sig(xr + hr)
        z = sig(xz + hz)
        nn_ = jnp.tanh(xn + r * hn)
        h_new = (1.0 - z) * nn_ + z * h
        m = (i * _TCHUNK + t) < lens_g
        h_out = jnp.where(m, h_new, h)
        return h_out, m

    def step(t, carry):
        h_a, h_b, hg_b = carry
        hg_a = hdot(h_a)
        xgt_b = xg_s[pl.ds(t * _B + _G, _G), :]
        hb_out, mb = gates(xgt_b, hg_b, h_b, t, lens[_G:])
        ys_s[pl.ds(t * _B + _G, _G), :] = hb_out
        csum_s[_G:, :] = csum_s[_G:, :] + jnp.where(mb, hb_out, 0.0)
        hg_b_next = hdot(hb_out)
        xgt_a = xg_s[pl.ds(t * _B, _G), :]
        ha_out, ma = gates(xgt_a, hg_a, h_a, t, lens[0:_G])
        ys_s[pl.ds(t * _B, _G), :] = ha_out
        csum_s[0:_G, :] = csum_s[0:_G, :] + jnp.where(ma, ha_out, 0.0)
        return ha_out, hb_out, hg_b_next

    _UNROLL = 8

    def stepu(u, carry):
        for k in range(_UNROLL):
            carry = step(_UNROLL * u + k, carry)
        return carry

    h0 = h_s[...]
    hg_b0 = hdot(h0[_G:])
    ha_f, hb_f, _ = lax.fori_loop(0, _TCHUNK // _UNROLL, stepu,
                                  (h0[0:_G], h0[_G:], hg_b0))
    h_s[...] = jnp.concatenate([ha_f, hb_f], axis=0)
    enc_ref[...] = ys_s[...].reshape(_TCHUNK, _B, _H_DIM).transpose(1, 0, 2)

    @pl.when(i == (_T // _TCHUNK) - 1)
    def _fin():
        csum_o[...] = csum_s[...]


def _tc_gru(ctx_in_tm, wx, wh, bg, lens):
    nchunks = _T // _TCHUNK
    return pl.pallas_call(
        _gru_body,
        grid=(nchunks,),
        in_specs=[
            pl.BlockSpec((_TCHUNK * _B, _IN_DIM), lambda i: (i, 0)),
            pl.BlockSpec((_IN_DIM, 3 * _H_DIM), lambda i: (0, 0)),
            pl.BlockSpec((_H_DIM, 3 * _H_DIM), lambda i: (0, 0)),
            pl.BlockSpec((1, 3 * _H_DIM), lambda i: (0, 0)),
            pl.BlockSpec((_B, 1), lambda i: (0, 0)),
        ],
        out_specs=[
            pl.BlockSpec((_B, _TCHUNK, _H_DIM), lambda i: (0, i, 0)),
            pl.BlockSpec((_B, _H_DIM), lambda i: (0, 0)),
        ],
        out_shape=(
            jax.ShapeDtypeStruct((_B, _T, _H_DIM), jnp.float32),
            jax.ShapeDtypeStruct((_B, _H_DIM), jnp.float32),
        ),
        scratch_shapes=[
            pltpu.VMEM((_B, _H_DIM), jnp.float32),
            pltpu.VMEM((_B, _H_DIM), jnp.float32),
            pltpu.VMEM((_TCHUNK * _B, 3 * _H_DIM), jnp.float32),
            pltpu.VMEM((_TCHUNK * _B, _H_DIM), jnp.float32),
        ],
    )(ctx_in_tm, wx, wh, bg, lens)


def kernel(packed_srcs, packed_srcs_positions, packed_tgts, packed_tgts_positions,
           packed_paths, packed_paths_positions, packed_ctx, ctx_lengths,
           focus_num_of_paths, path_emb, src_tgt_emb, pos_emb, ctx_table,
           W_path, b_path, W_mix, b_mix, Wx, Wh, bg):
    ctx_rows_tm = _sc_ctx_gather(packed_ctx.T.reshape(-1), ctx_table)

    src_sum, tgt_sum, pth_sum = _sc_path_sums(
        packed_srcs.reshape(-1), packed_tgts.reshape(-1),
        packed_paths.reshape(-1), src_tgt_emb, path_emb)

    lens = ctx_lengths.reshape(_B, 1)
    enc, csum = _tc_gru(ctx_rows_tm, Wx.astype(jnp.bfloat16),
                        Wh.astype(jnp.bfloat16), bg.reshape(1, -1), lens)

    dlens = (focus_num_of_paths + ctx_lengths).astype(jnp.float32).reshape(_B, 1)
    mixed, h = _tc_fuse(
        src_sum, tgt_sum, pth_sum,
        packed_srcs, packed_srcs_positions,
        packed_tgts, packed_tgts_positions,
        packed_paths, packed_paths_positions,
        src_tgt_emb[0:1], path_emb[0:1], pos_emb,
        W_path, b_path.reshape(1, -1), W_mix, b_mix.reshape(1, -1),
        csum, dlens)

    return (mixed.reshape(_B, _P, _H_DIM), enc, h)


# speedup vs baseline: 1.2999x; 1.0169x over previous
import functools

import jax
import jax.numpy as jnp
from jax import lax
from jax.experimental import pallas as pl
from jax.experimental.pallas import tpu as pltpu
from jax.experimental.pallas import tpu_sc as plsc

_B = 16
_P = 128
_T = 512
_SRC_LEN = 8
_PATH_LEN = 12
_IN_DIM = 128
_H_DIM = 256
_NUM_LAYERS = 2
_NW = 32


def _sc_ctx_gather(ctx_i, ctx_t):
    ctx_per_w = (_B * _T) // _NW
    mesh = plsc.VectorSubcoreMesh(core_axis_name="c", subcore_axis_name="s")

    @functools.partial(
        pl.kernel,
        out_type=jax.ShapeDtypeStruct((_B * _T, _IN_DIM), jnp.float32),
        mesh=mesh,
        scratch_types=[
            pltpu.VMEM((ctx_per_w,), jnp.int32),
            pltpu.VMEM((ctx_per_w, _IN_DIM), jnp.float32),
            pltpu.SemaphoreType.DMA,
        ],
    )
    def k(ctxi_r, ctxt_r, ctx_o, idxc, rows, sem):
        wid = lax.axis_index("s") * 2 + lax.axis_index("c")
        cb = wid * ctx_per_w
        pltpu.sync_copy(ctxi_r.at[pl.ds(cb, ctx_per_w)], idxc)
        pltpu.async_copy(ctxt_r.at[idxc], rows, sem).wait()
        pltpu.sync_copy(rows, ctx_o.at[pl.ds(cb, ctx_per_w)])

    return k(ctx_i, ctx_t)


def _sc_path_sums(srcs_i, tgts_i, paths_i, st_t, path_t):
    n_rows = _B * _P
    rows_per_w = n_rows // _NW
    mesh = plsc.VectorSubcoreMesh(core_axis_name="c", subcore_axis_name="s")

    @functools.partial(
        pl.kernel,
        out_type=(
            jax.ShapeDtypeStruct((n_rows, _IN_DIM), jnp.float32),
            jax.ShapeDtypeStruct((n_rows, _IN_DIM), jnp.float32),
            jax.ShapeDtypeStruct((n_rows, _IN_DIM), jnp.float32),
        ),
        mesh=mesh,
        scratch_types=[
            pltpu.VMEM((16 * _SRC_LEN,), jnp.int32),
            pltpu.VMEM((16 * _PATH_LEN,), jnp.int32),
            pltpu.VMEM((16 * _PATH_LEN, _IN_DIM), jnp.float32),
            pltpu.VMEM((16, _IN_DIM), jnp.float32),
            pltpu.SemaphoreType.DMA,
        ],
    )
    def k(srcs_r, tgts_r, paths_r, st_r, path_r,
          src_o, tgt_o, pth_o, idx8, idx12, rows, acc, sem):
        wid = lax.axis_index("s") * 2 + lax.axis_index("c")

        def reduce_rep(idx_hbm, table, out_hbm, G, idx_v):
            base = wid * rows_per_w

            def chunk(c, carry):
                rb = base + c * 16
                pltpu.sync_copy(idx_hbm.at[pl.ds(rb * G, 16 * G)], idx_v)
                pltpu.async_copy(table.at[idx_v],
                                 rows.at[pl.ds(0, 16 * G)], sem).wait()

                def rowfn(r, carry2):
                    for v in range(_IN_DIM // 16):
                        s = rows[r * G, pl.ds(v * 16, 16)]
                        for j in range(1, G):
                            s = s + rows[r * G + j, pl.ds(v * 16, 16)]
                        acc[r, pl.ds(v * 16, 16)] = s
                    return carry2

                lax.fori_loop(0, 16, rowfn, 0)
                pltpu.sync_copy(acc, out_hbm.at[pl.ds(rb, 16)])
                return carry

            lax.fori_loop(0, rows_per_w // 16, chunk, 0)

        reduce_rep(srcs_r, st_r, src_o, _SRC_LEN, idx8)
        reduce_rep(tgts_r, st_r, tgt_o, _SRC_LEN, idx8)
        reduce_rep(paths_r, path_r, pth_o, _PATH_LEN, idx12)

    return k(srcs_i, tgts_i, paths_i, st_t, path_t)


def _fuse_body(ss, ts, ps, stok, spos, ttok, tpos, ptok, ppos,
               r0st, r0p, pose, wp, bp, wm, bm, csum, dlens, mixed_o, h_o):
    n = _B * _P
    pos_tab = pose[...].astype(jnp.bfloat16)

    def rep(sum_ref, tok_ref, pos_ref, row0_ref, G):
        tok = tok_ref[...]
        posi = pos_ref[...]
        mask = (tok != 0).astype(jnp.float32)
        cnt = mask.sum(axis=1, keepdims=True)
        cnt0 = G - cnt
        iota32 = lax.broadcasted_iota(jnp.int32, (n, 32), 1)
        hist = jnp.zeros((n, 32), jnp.float32)
        for j in range(G):
            hist = hist + (posi[:, j:j + 1] == iota32).astype(jnp.float32) * mask[:, j:j + 1]
        pos_contrib = jnp.dot(hist.astype(jnp.bfloat16), pos_tab,
                              preferred_element_type=jnp.float32)
        return (sum_ref[...] - cnt0 * row0_ref[...] + pos_contrib) / jnp.maximum(cnt, 1.0)

    sr = rep(ss, stok, spos, r0st, _SRC_LEN)
    tr = rep(ts, ttok, tpos, r0st, _SRC_LEN)
    pr = rep(ps, ptok, ppos, r0p, _PATH_LEN)

    wp_v = wp[...].astype(jnp.bfloat16)
    ep = jnp.tanh(
        jnp.dot(sr.astype(jnp.bfloat16), wp_v[0:128], preferred_element_type=jnp.float32)
        + jnp.dot(tr.astype(jnp.bfloat16), wp_v[128:256], preferred_element_type=jnp.float32)
        + jnp.dot(pr.astype(jnp.bfloat16), wp_v[256:384], preferred_element_type=jnp.float32)
        + bp[...])
    mixed_o[...] = jnp.tanh(
        jnp.dot(ep.astype(jnp.bfloat16), wm[...].astype(jnp.bfloat16),
                preferred_element_type=jnp.float32) + bm[...])
    gids = lax.broadcasted_iota(jnp.int32, (_B, n), 1) // _P
    sel = (gids == lax.broadcasted_iota(jnp.int32, (_B, n), 0)).astype(jnp.float32)
    psum = jnp.dot(sel, ep, preferred_element_type=jnp.float32)
    hf = (psum + csum[...]) / dlens[...]
    h_o[...] = jnp.broadcast_to(hf[None], (_NUM_LAYERS, _B, _H_DIM))


def _tc_fuse(src_sum, tgt_sum, pth_sum, stok, spos, ttok, tpos, ptok, ppos,
             r0st, r0p, pos_emb, wp, bp, wm, bm, csum, dlens):
    n = _B * _P
    return pl.pallas_call(
        _fuse_body,
        out_shape=(
            jax.ShapeDtypeStruct((n, _H_DIM), jnp.float32),
            jax.ShapeDtypeStruct((_NUM_LAYERS, _B, _H_DIM), jnp.float32),
        ),
    )(src_sum, tgt_sum, pth_sum, stok, spos, ttok, tpos, ptok, ppos,
      r0st, r0p, pos_emb, wp, bp, wm, bm, csum, dlens)


_TCHUNK = 64


def _gru_body(ctx_ref, wx_ref, wh_ref, bg_ref, len_ref,
              enc_ref, csum_o, h_s, csum_s, xg_s, ys_s):
    i = pl.program_id(0)

    @pl.when(i == 0)
    def _init():
        h_s[...] = jnp.zeros_like(h_s)
        csum_s[...] = jnp.zeros_like(csum_s)

    x = ctx_ref[...].astype(jnp.bfloat16)
    xg_s[...] = jnp.dot(x, wx_ref[...], preferred_element_type=jnp.float32) + bg_ref[...]
    lens = len_ref[...]
    wh = wh_ref[...]
    _G = _B // 2
    def hdot(h):
        return jnp.dot(h.astype(jnp.bfloat16), wh, preferred_element_type=jnp.float32)

    sig = jax.nn.sigmoid

    def gates(xgt, hg, h, t, lens_g):
        xr = xgt[:, 0:_H_DIM]
        xz = xgt[:, _H_DIM:2 * _H_DIM]
        xn = xgt[:, 2 * _H_DIM:]
        hr = hg[:, 0:_H_DIM]
        hz = hg[:, _H_DIM:2 * _H_DIM]
        hn = hg[:, 2 * _H_DIM:]
        r = sig(xr + hr)
        z = sig(xz + hz)
        nn_ = jnp.tanh(xn + r * hn)
        h_new = (1.0 - z) * nn_ + z * h
        m = (i * _TCHUNK + t) < lens_g
        h_out = jnp.where(m, h_new, h)
        return h_out, m

    def step(t, carry):
        h_a, h_b, hg_b = carry
        hg_a = hdot(h_a)
        xgt_b = xg_s[pl.ds(t * _B + _G, _G), :]
        hb_out, mb = gates(xgt_b, hg_b, h_b, t, lens[_G:])
        ys_s[pl.ds(t * _B + _G, _G), :] = hb_out
        csum_s[_G:, :] = csum_s[_G:, :] + jnp.where(mb, hb_out, 0.0)
        hg_b_next = hdot(hb_out)
        xgt_a = xg_s[pl.ds(t * _B, _G), :]
        ha_out, ma = gates(xgt_a, hg_a, h_a, t, lens[0:_G])
        ys_s[pl.ds(t * _B, _G), :] = ha_out
        csum_s[0:_G, :] = csum_s[0:_G, :] + jnp.where(ma, ha_out, 0.0)
        return ha_out, hb_out, hg_b_next

    _UNROLL = 16

    def stepu(u, carry):
        for k in range(_UNROLL):
            carry = step(_UNROLL * u + k, carry)
        return carry

    h0 = h_s[...]
    hg_b0 = hdot(h0[_G:])
    ha_f, hb_f, _ = lax.fori_loop(0, _TCHUNK // _UNROLL, stepu,
                                  (h0[0:_G], h0[_G:], hg_b0))
    h_s[...] = jnp.concatenate([ha_f, hb_f], axis=0)
    enc_ref[...] = ys_s[...].reshape(_TCHUNK, _B, _H_DIM).transpose(1, 0, 2)

    @pl.when(i == (_T // _TCHUNK) - 1)
    def _fin():
        csum_o[...] = csum_s[...]


def _tc_gru(ctx_in_tm, wx, wh, bg, lens):
    nchunks = _T // _TCHUNK
    return pl.pallas_call(
        _gru_body,
        grid=(nchunks,),
        in_specs=[
            pl.BlockSpec((_TCHUNK * _B, _IN_DIM), lambda i: (i, 0)),
            pl.BlockSpec((_IN_DIM, 3 * _H_DIM), lambda i: (0, 0)),
            pl.BlockSpec((_H_DIM, 3 * _H_DIM), lambda i: (0, 0)),
            pl.BlockSpec((1, 3 * _H_DIM), lambda i: (0, 0)),
            pl.BlockSpec((_B, 1), lambda i: (0, 0)),
        ],
        out_specs=[
            pl.BlockSpec((_B, _TCHUNK, _H_DIM), lambda i: (0, i, 0)),
            pl.BlockSpec((_B, _H_DIM), lambda i: (0, 0)),
        ],
        out_shape=(
            jax.ShapeDtypeStruct((_B, _T, _H_DIM), jnp.float32),
            jax.ShapeDtypeStruct((_B, _H_DIM), jnp.float32),
        ),
        scratch_shapes=[
            pltpu.VMEM((_B, _H_DIM), jnp.float32),
            pltpu.VMEM((_B, _H_DIM), jnp.float32),
            pltpu.VMEM((_TCHUNK * _B, 3 * _H_DIM), jnp.float32),
            pltpu.VMEM((_TCHUNK * _B, _H_DIM), jnp.float32),
        ],
    )(ctx_in_tm, wx, wh, bg, lens)


def kernel(packed_srcs, packed_srcs_positions, packed_tgts, packed_tgts_positions,
           packed_paths, packed_paths_positions, packed_ctx, ctx_lengths,
           focus_num_of_paths, path_emb, src_tgt_emb, pos_emb, ctx_table,
           W_path, b_path, W_mix, b_mix, Wx, Wh, bg):
    ctx_rows_tm = _sc_ctx_gather(packed_ctx.T.reshape(-1), ctx_table)

    src_sum, tgt_sum, pth_sum = _sc_path_sums(
        packed_srcs.reshape(-1), packed_tgts.reshape(-1),
        packed_paths.reshape(-1), src_tgt_emb, path_emb)

    lens = ctx_lengths.reshape(_B, 1)
    enc, csum = _tc_gru(ctx_rows_tm, Wx.astype(jnp.bfloat16),
                        Wh.astype(jnp.bfloat16), bg.reshape(1, -1), lens)

    dlens = (focus_num_of_paths + ctx_lengths).astype(jnp.float32).reshape(_B, 1)
    mixed, h = _tc_fuse(
        src_sum, tgt_sum, pth_sum,
        packed_srcs, packed_srcs_positions,
        packed_tgts, packed_tgts_positions,
        packed_paths, packed_paths_positions,
        src_tgt_emb[0:1], path_emb[0:1], pos_emb,
        W_path, b_path.reshape(1, -1), W_mix, b_mix.reshape(1, -1),
        csum, dlens)

    return (mixed.reshape(_B, _P, _H_DIM), enc, h)
